# baseline (speedup 1.0000x reference)
import functools

import jax
import jax.numpy as jnp
from jax import lax
from jax.experimental import pallas as pl
from jax.experimental.pallas import tpu as pltpu
from jax.experimental.pallas import tpu_sc as plsc

VOCAB_ = 1000000
SEQ_ = 200
BATCH_ = 4096
EMB_ = 32

NUM_WORKERS = 32
TOTAL = BATCH_ * SEQ_
BLK = 128
UNITS_PER_W = TOTAL // BLK // NUM_WORKERS
BLOCKS_PER_SEQ = BATCH_ // BLK
KGROUPS = BLK // 16

NA_MAIN = 488
A_MAIN_ROWS = 16 * NA_MAIN * 128


def _sc_lookup(tok_t, emb_t, tail):
  mesh = plsc.VectorSubcoreMesh(core_axis_name="c", subcore_axis_name="s")

  @functools.partial(
      pl.kernel,
      out_type=jax.ShapeDtypeStruct((SEQ_, EMB_, BATCH_), jnp.float32),
      mesh=mesh,
      scratch_types=[
          pltpu.VMEM((UNITS_PER_W * BLK,), jnp.int32),
          pltpu.VMEM((BLK,), jnp.int32),
          pltpu.VMEM((BLK,), jnp.int32),
          pltpu.VMEM((EMB_, 128), jnp.float32),
          pltpu.VMEM((EMB_, 128), jnp.float32),
          pltpu.VMEM((EMB_, 128), jnp.float32),
          pltpu.VMEM((EMB_, 128), jnp.float32),
          pltpu.VMEM((BLK, 128), jnp.float32),
          pltpu.VMEM((BLK, 128), jnp.float32),
          pltpu.VMEM((EMB_, BLK), jnp.float32),
          pltpu.VMEM((EMB_, BLK), jnp.float32),
          pltpu.VMEM((16,), jnp.int32),
          pltpu.VMEM((16,), jnp.int32),
          pltpu.VMEM((16, 128), jnp.float32),
          pltpu.HBM((VOCAB_ // 4, 128), jnp.float32),
          pltpu.HBM((16,), jnp.int32),
          pltpu.HBM((16,), jnp.int32),
          pltpu.SemaphoreType.DMA,
          pltpu.SemaphoreType.DMA,
          pltpu.SemaphoreType.DMA,
          pltpu.SemaphoreType.DMA,
          pltpu.SemaphoreType.DMA,
          pltpu.SemaphoreType.DMA,
          pltpu.SemaphoreType.DMA,
          pltpu.SemaphoreType.DMA,
      ],
      compiler_params=pltpu.CompilerParams(needs_layout_passes=False),
  )
  def body(tok_hbm, emb_hbm, tail_hbm, out_hbm, idx_all, ix0, ix1, ai0,
           ai1, ao0, ao1, g0, g1, t0, t1, fw, fp, tailv, staging, flag0,
           flag1, ag0, ag1, aw0, aw1, gs0, gs1, ws0, ws1):
    flags = (flag0, flag1)
    idxb = (ix0, ix1)
    ain = (ai0, ai1)
    aout = (ao0, ao1)
    g = (g0, g1)
    tb_ = (t0, t1)
    agsem = (ag0, ag1)
    awsem = (aw0, aw1)
    gsem = (gs0, gs1)
    wsem = (ws0, ws1)
    cid = lax.axis_index("c")
    sid = lax.axis_index("s")
    wid = sid * 2 + cid
    ubase = wid * UNITS_PER_W

    for f in range(2):

      @pl.when((sid == 0) & (cid == f))
      def _():
        fw[pl.ds(0, 16)] = jnp.zeros((16,), jnp.int32)
        pltpu.sync_copy(fw, flags[f])

    pltpu.sync_copy(
        tok_hbm.at[pl.ds(pl.multiple_of(ubase * BLK, 1024),
                         UNITS_PER_W * BLK)], idx_all)

    iota = lax.iota(jnp.int32, 16)
    rowvec = [iota + 16 * k for k in range(KGROUPS)]
    cvecs = [jnp.full((16,), c, jnp.int32) for c in range(EMB_)]

    def a_row0(u):
      return pl.multiple_of((sid * NA_MAIN + u) * 128, 128)

    half_iota = [iota + 16 * h for h in range(2)]

    def a_transpose(b):
      for q in range(32):
        vals = [
            plsc.load_gather(
                ain[b],
                [half_iota[k % 2],
                 jnp.full((16,), 4 * q + k // 2, jnp.int32)])
            for k in range(8)
        ]
        for k in range(8):
          aout[b][q, pl.ds(16 * k, 16)] = vals[k]

    def a_start(u, b):
      pltpu.async_copy(emb_hbm.at[:, pl.ds(a_row0(u), 128)], ain[b],
                       agsem[b])

    def a_wait_in(u, b):
      pltpu.make_async_copy(emb_hbm.at[:, pl.ds(a_row0(u), 128)], ain[b],
                            agsem[b]).wait()

    def a_pr0(u):
      return pl.multiple_of((sid * NA_MAIN + u) * 32, 32)

    def a_wait_out(u, b):
      pltpu.make_async_copy(aout[b], staging.at[pl.ds(a_pr0(u), 32)],
                            awsem[b]).wait()

    a_start(0, 0)
    a_start(1, 1)

    def a_step(it, carry):
      for b in range(2):
        u = 2 * it + b

        @pl.when(u >= 2)
        def _():
          a_wait_out(u, b)

        a_wait_in(u, b)
        a_transpose(b)

        @pl.when(u + 2 < NA_MAIN)
        def _():
          a_start(u + 2, b)

        pltpu.async_copy(aout[b], staging.at[pl.ds(a_pr0(u), 32)],
                         awsem[b])
      return carry

    lax.fori_loop(0, NA_MAIN // 2, a_step, 0)
    for b in range(2):
      a_wait_out(0, b)

    for e in range(4):

      @pl.when(sid == e)
      def _():
        r0 = A_MAIN_ROWS + e * 128
        pltpu.sync_copy(emb_hbm.at[:, pl.ds(r0, 128)], ain[0])
        a_transpose(0)
        pltpu.sync_copy(aout[0], staging.at[pl.ds(r0 // 4, 32)])

    @pl.when(sid == 4)
    def _():
      pltpu.sync_copy(tail_hbm, tailv)
      pltpu.sync_copy(tailv, staging.at[pl.ds(VOCAB_ // 4 - 16, 16)])

    plsc.subcore_barrier()

    for f in range(2):

      @pl.when((sid == 0) & (cid == f))
      def _():
        fw[pl.ds(0, 16)] = jnp.full((16,), 1, jnp.int32)
        pltpu.sync_copy(fw, flags[f])

    def poll_cond(seen):
      return seen == 0

    for f in range(2):

      @pl.when(cid == 1 - f)
      def _():

        def poll_body(seen):
          pltpu.sync_copy(flags[f], fp)
          v = fp[pl.ds(0, 16)]
          return v[0]

        lax.while_loop(poll_cond, poll_body, jnp.int32(0))

    def c_start(u, b):
      off = u * BLK
      for k in range(KGROUPS):
        v = idx_all[pl.ds(off + 16 * k, 16)]
        idxb[b][pl.ds(16 * k, 16)] = lax.shift_right_logical(v, 2)
      pltpu.async_copy(staging.at[idxb[b]], g[b], gsem[b])

    def c_wait_g(u, b):
      pltpu.make_async_copy(staging.at[idxb[b]], g[b], gsem[b]).wait()

    def c_transpose(u, b):
      off = u * BLK
      for k in range(KGROUPS):
        v = idx_all[pl.ds(off + 16 * k, 16)]
        colbase = lax.shift_left(jnp.bitwise_and(v, 3), 5)
        vals = [plsc.load_gather(g[b], [rowvec[k], colbase + cvecs[c]])
                for c in range(EMB_)]
        for c in range(EMB_):
          tb_[b][c, pl.ds(16 * k, 16)] = vals[c]

    def out_ref(gu):
      s = gu // BLOCKS_PER_SEQ
      tbk = gu % BLOCKS_PER_SEQ
      return out_hbm.at[s, :, pl.ds(pl.multiple_of(tbk * BLK, BLK), BLK)]

    def wait_write(gu, b):
      pltpu.make_async_copy(tb_[b], out_ref(gu), wsem[b]).wait()

    c_start(0, 0)
    c_start(1, 1)

    def c_step(it, carry):
      for b in range(2):
        u = 2 * it + b
        gu = ubase + u

        @pl.when(u >= 2)
        def _():
          wait_write(gu, b)

        c_wait_g(u, b)
        c_transpose(u, b)

        @pl.when(u + 2 < UNITS_PER_W)
        def _():
          c_start(u + 2, b)

        pltpu.async_copy(tb_[b], out_ref(gu), wsem[b])
      return carry

    lax.fori_loop(0, UNITS_PER_W // 2, c_step, 0)
    for b in range(2):
      wait_write(ubase, b)

  return body(tok_t, emb_t, tail)


def kernel(tokens, embedding):
  tok_t = jnp.transpose(tokens).reshape(TOTAL).astype(jnp.int32)
  emb_t = jnp.transpose(embedding)
  tail = lax.slice(embedding, (VOCAB_ - 64, 0),
                   (VOCAB_, EMB_)).reshape(16, 128)
  r = _sc_lookup(tok_t, emb_t, tail)
  return jnp.transpose(r, (2, 0, 1))


# speedup vs baseline: 1.8366x; 1.8366x over previous
import functools

import jax
import jax.numpy as jnp
from jax import lax
from jax.experimental import pallas as pl
from jax.experimental.pallas import tpu as pltpu
from jax.experimental.pallas import tpu_sc as plsc

VOCAB_ = 1000000
SEQ_ = 200
BATCH_ = 4096
EMB_ = 32

NUM_WORKERS = 32
PACK = 128 // EMB_
TOTAL = BATCH_ * SEQ_
BLK = 128
UNITS = TOTAL // BLK
UNITS_PER_W = UNITS // NUM_WORKERS
BLOCKS_PER_SEQ = BATCH_ // BLK
KGROUPS = BLK // 16


def _sc_lookup(tok_t, resh):
  mesh = plsc.VectorSubcoreMesh(core_axis_name="c", subcore_axis_name="s")

  @functools.partial(
      pl.kernel,
      out_type=jax.ShapeDtypeStruct((SEQ_, EMB_, BATCH_), jnp.float32),
      mesh=mesh,
      scratch_types=[
          pltpu.VMEM((UNITS_PER_W * BLK,), jnp.int32),
          pltpu.VMEM((BLK,), jnp.int32),
          pltpu.VMEM((BLK,), jnp.int32),
          pltpu.VMEM((BLK, 128), jnp.float32),
          pltpu.VMEM((BLK, 128), jnp.float32),
          pltpu.VMEM((EMB_, BLK), jnp.float32),
          pltpu.VMEM((EMB_, BLK), jnp.float32),
          pltpu.SemaphoreType.DMA,
          pltpu.SemaphoreType.DMA,
          pltpu.SemaphoreType.DMA,
          pltpu.SemaphoreType.DMA,
      ],
      compiler_params=pltpu.CompilerParams(needs_layout_passes=False),
  )
  def body(tok_hbm, tab_hbm, out_hbm, idx_all, i40, i41, g0, g1, t0, t1,
           gs0, gs1, ws0, ws1):
    idx4 = (i40, i41)
    g = (g0, g1)
    tb_ = (t0, t1)
    gsem = (gs0, gs1)
    wsem = (ws0, ws1)
    wid = lax.axis_index("s") * 2 + lax.axis_index("c")
    ubase = wid * UNITS_PER_W

    pltpu.sync_copy(tok_hbm.at[pl.ds(ubase * BLK, UNITS_PER_W * BLK)],
                    idx_all)

    iota = lax.iota(jnp.int32, 16)
    rowvec = [iota + 16 * k for k in range(KGROUPS)]

    def compute_idx(u, b):
      off = u * BLK
      for k in range(KGROUPS):
        v = idx_all[pl.ds(off + 16 * k, 16)]
        idx4[b][pl.ds(16 * k, 16)] = lax.shift_right_logical(v, 2)

    def start_gather(u, b):
      compute_idx(u, b)
      pltpu.async_copy(tab_hbm.at[idx4[b]], g[b], gsem[b])

    def wait_gather(b):
      pltpu.make_async_copy(tab_hbm.at[idx4[b]], g[b], gsem[b]).wait()

    def transpose(u, b):
      off = u * BLK
      for k in range(KGROUPS):
        v = idx_all[pl.ds(off + 16 * k, 16)]
        colbase = lax.shift_left(jnp.bitwise_and(v, PACK - 1), 5)
        vals = [plsc.load_gather(g[b], [rowvec[k], colbase + c])
                for c in range(EMB_)]
        for c in range(EMB_):
          tb_[b][c, pl.ds(16 * k, 16)] = vals[c]

    def out_ref(gu):
      s = gu // BLOCKS_PER_SEQ
      tbk = gu % BLOCKS_PER_SEQ
      return out_hbm.at[s, :, pl.ds(tbk * BLK, BLK)]

    def wait_write(gu, b):
      pltpu.make_async_copy(tb_[b], out_ref(gu), wsem[b]).wait()

    start_gather(0, 0)
    start_gather(1, 1)

    def step(it, carry):
      for b in range(2):
        u = 2 * it + b
        gu = ubase + u

        @pl.when(u >= 2)
        def _():
          wait_write(gu, b)

        wait_gather(b)
        transpose(u, b)
        j = u + 2

        @pl.when(j < UNITS_PER_W)
        def _():
          start_gather(j, b)

        pltpu.async_copy(tb_[b], out_ref(gu), wsem[b])
      return carry

    lax.fori_loop(0, UNITS_PER_W // 2, step, 0)
    for b in range(2):
      wait_write(ubase, b)

  return body(tok_t, resh)


def kernel(tokens, embedding):
  tok_t = jnp.transpose(tokens).reshape(TOTAL).astype(jnp.int32)
  resh = embedding.reshape(VOCAB_ // PACK, 128)
  r = _sc_lookup(tok_t, resh)
  return jnp.transpose(r, (2, 0, 1))
